# baseline (speedup 1.0000x reference)
import jax
import jax.numpy as jnp
from jax.experimental import pallas as pl
from jax.experimental.pallas import tpu as pltpu

_NT = 27
_H = 128
_SEQ = 20
_NE = _NT * 3


def _gkt_kernel(idx3c_ref, taskt_ref, p_ref, q_ref, emb_ref, wet_ref,
                wat_ref, whht_ref, bih_ref, bhh_ref, pw_ref, pb_ref,
                outs_ref, hid_ref, hist_ref):
    B = idx3c_ref.shape[0]
    f32 = jnp.float32

    gi_tab = jnp.dot(emb_ref[...], wet_ref[...],
                     preferred_element_type=f32) + bih_ref[...]
    wat_s = wat_ref[...] * jnp.float32(1.0 / 28.0)
    whht = whht_ref[...]
    bhh = bhh_ref[...]
    pw = pw_ref[...]
    pb = pb_ref[0, 0]

    iota81 = jax.lax.broadcasted_iota(jnp.int32, (B, _NE), 1)
    iota27l = jax.lax.broadcasted_iota(jnp.int32, (_NT, B), 0)

    zero_plane = jnp.zeros((B, _H), f32)
    S = zero_plane
    dT = jnp.full((_NT, B), pb, f32)
    inv28 = f32(1.0 / 28.0)

    for t in range(_SEQ):
        idx3c = idx3c_ref[:, t:t + 1]

        prev_h = zero_plane
        if t > 0:
            pc = p_ref[:, t:t + 1]

            def _gather(tp, acc):
                return acc + jnp.where(pc == tp, hist_ref[tp], f32(0.0))

            prev_h = jax.lax.fori_loop(0, t, _gather, zero_plane)

        oh81 = (idx3c == iota81).astype(f32)
        gi_e = jnp.dot(oh81, gi_tab, preferred_element_type=f32)

        gi = gi_e + jnp.dot(S + prev_h, wat_s, preferred_element_type=f32)
        gh = jnp.dot(prev_h, whht, preferred_element_type=f32) + bhh

        r = jax.nn.sigmoid(gi[:, :_H] + gh[:, :_H])
        z = jax.nn.sigmoid(gi[:, _H:2 * _H] + gh[:, _H:2 * _H])
        nn = jnp.tanh(gi[:, 2 * _H:] + r * gh[:, 2 * _H:])
        new_h = nn + z * (prev_h - nn)

        hist_ref[t] = new_h
        S = S + new_h - prev_h

        lnewT = jax.lax.dot_general(pw, new_h, (((1,), (1,)), ((), ())),
                                    preferred_element_type=f32) + pb
        taskt = taskt_ref[t:t + 1, :]
        dT = jnp.where(iota27l == taskt, lnewT, dT)
        outs_ref[t] = dT

    for n in range(_NT):
        qc = q_ref[:, n:n + 1]

        def _recon(t, hn):
            return jnp.where(qc == t, hist_ref[t], hn)

        hid_ref[n] = jax.lax.fori_loop(0, _SEQ, _recon, zero_plane)


def kernel(task_seq, status_seq, emb_table, gru_Wih, gru_Whh, gru_bih,
           gru_bhh, pred_W, pred_b):
    B = task_seq.shape[0]
    f32 = jnp.float32

    idx3 = task_seq * 3 + status_seq
    taskT = jnp.transpose(task_seq)

    tt = jnp.arange(_SEQ, dtype=jnp.int32)
    eq = task_seq[:, :, None] == task_seq[:, None, :]
    tril = tt[None, :, None] > tt[None, None, :]
    p = jnp.max(jnp.where(eq & tril, tt[None, None, :], -1), axis=2)
    eqn = task_seq[:, None, :] == jnp.arange(_NT, dtype=jnp.int32)[None, :, None]
    q = jnp.max(jnp.where(eqn, tt[None, None, :], -1), axis=2)

    wet = jnp.transpose(gru_Wih[:, :_H])
    wat = jnp.transpose(gru_Wih[:, _H:])
    whht = jnp.transpose(gru_Whh)
    bih = gru_bih.reshape(1, 3 * _H).astype(f32)
    bhh = gru_bhh.reshape(1, 3 * _H).astype(f32)
    pw = pred_W.reshape(1, _H).astype(f32)
    pb = pred_b.reshape(1, 1).astype(f32)

    outs_raw, hid_raw = pl.pallas_call(
        _gkt_kernel,
        out_shape=[
            jax.ShapeDtypeStruct((_SEQ, _NT, B), f32),
            jax.ShapeDtypeStruct((_NT, B, _H), f32),
        ],
        scratch_shapes=[pltpu.VMEM((_SEQ, B, _H), f32)],
    )(idx3, taskT, p, q, emb_table.astype(f32), wet, wat, whht,
      bih, bhh, pw, pb)

    outs = jnp.transpose(outs_raw, (2, 0, 1))
    hidden = jnp.transpose(hid_raw, (1, 0, 2))
    return outs, hidden


# speedup vs baseline: 1.1494x; 1.1494x over previous
import jax
import jax.numpy as jnp
from jax.experimental import pallas as pl
from jax.experimental.pallas import tpu as pltpu

_NT = 27
_H = 128
_SEQ = 20
_NE = _NT * 3
_GRID = 4


def _gkt_kernel(idx3c_ref, taskt_ref, p_ref, q_ref, emb_ref, wet_ref,
                wat_ref, whht_ref, bih_ref, bhh_ref, pw_ref, pb_ref,
                outs_ref, hid_ref, hist_ref):
    B = idx3c_ref.shape[0]
    f32 = jnp.float32

    gi_tab = jnp.dot(emb_ref[...], wet_ref[...],
                     preferred_element_type=f32) + bih_ref[...]
    wat_s = wat_ref[...] * jnp.float32(1.0 / 28.0)
    whht = whht_ref[...]
    bhh = bhh_ref[...]
    pw = pw_ref[...]
    pb = pb_ref[0, 0]

    iota81 = jax.lax.broadcasted_iota(jnp.int32, (B, _NE), 1)
    iota27l = jax.lax.broadcasted_iota(jnp.int32, (_NT, B), 0)

    zero_plane = jnp.zeros((B, _H), f32)
    S = zero_plane
    dT = jnp.full((_NT, B), pb, f32)

    for t in range(_SEQ):
        idx3c = idx3c_ref[:, t:t + 1]

        prev_h = zero_plane
        if t > 0:
            pc = p_ref[:, t:t + 1]
            for tp in range(t):
                prev_h = prev_h + jnp.where(pc == tp, hist_ref[tp], f32(0.0))

        oh81 = (idx3c == iota81).astype(f32)
        gi_e = jnp.dot(oh81, gi_tab, preferred_element_type=f32)

        gi = gi_e + jnp.dot(S + prev_h, wat_s, preferred_element_type=f32)
        gh = jnp.dot(prev_h, whht, preferred_element_type=f32) + bhh

        r = jax.nn.sigmoid(gi[:, :_H] + gh[:, :_H])
        z = jax.nn.sigmoid(gi[:, _H:2 * _H] + gh[:, _H:2 * _H])
        nn = jnp.tanh(gi[:, 2 * _H:] + r * gh[:, 2 * _H:])
        new_h = nn + z * (prev_h - nn)

        hist_ref[t] = new_h
        S = S + new_h - prev_h

        lnewT = jax.lax.dot_general(pw, new_h, (((1,), (1,)), ((), ())),
                                    preferred_element_type=f32) + pb
        taskt = taskt_ref[t:t + 1, :]
        dT = jnp.where(iota27l == taskt, lnewT, dT)
        outs_ref[t] = dT

    dep = jnp.zeros((B, 1), jnp.int32)
    for n in range(_NT):
        qc = q_ref[:, n:n + 1] + dep
        hn = zero_plane
        for t in range(_SEQ):
            hn = jnp.where(qc == t, hist_ref[t], hn)
        hid_ref[n] = hn
        dep = (hn[:, :1] == f32(jnp.inf)).astype(jnp.int32)


def kernel(task_seq, status_seq, emb_table, gru_Wih, gru_Whh, gru_bih,
           gru_bhh, pred_W, pred_b):
    B = task_seq.shape[0]
    Bblk = B // _GRID
    f32 = jnp.float32

    idx3 = task_seq * 3 + status_seq
    taskT = jnp.transpose(task_seq)

    tt = jnp.arange(_SEQ, dtype=jnp.int32)
    eq = task_seq[:, :, None] == task_seq[:, None, :]
    tril = tt[None, :, None] > tt[None, None, :]
    p = jnp.max(jnp.where(eq & tril, tt[None, None, :], -1), axis=2)
    eqn = task_seq[:, None, :] == jnp.arange(_NT, dtype=jnp.int32)[None, :, None]
    q = jnp.max(jnp.where(eqn, tt[None, None, :], -1), axis=2)

    wet = jnp.transpose(gru_Wih[:, :_H])
    wat = jnp.transpose(gru_Wih[:, _H:])
    whht = jnp.transpose(gru_Whh)
    bih = gru_bih.reshape(1, 3 * _H).astype(f32)
    bhh = gru_bhh.reshape(1, 3 * _H).astype(f32)
    pw = pred_W.reshape(1, _H).astype(f32)
    pb = pred_b.reshape(1, 1).astype(f32)

    full = lambda *shape: pl.BlockSpec(shape, lambda i: (0,) * len(shape))

    outs_raw, hid_raw = pl.pallas_call(
        _gkt_kernel,
        grid=(_GRID,),
        in_specs=[
            pl.BlockSpec((Bblk, _SEQ), lambda i: (i, 0)),
            pl.BlockSpec((_SEQ, Bblk), lambda i: (0, i)),
            pl.BlockSpec((Bblk, _SEQ), lambda i: (i, 0)),
            pl.BlockSpec((Bblk, _NT), lambda i: (i, 0)),
            full(_NE, _H), full(_H, 3 * _H), full(_H, 3 * _H),
            full(_H, 3 * _H), full(1, 3 * _H), full(1, 3 * _H),
            full(1, _H), full(1, 1),
        ],
        out_specs=[
            pl.BlockSpec((_SEQ, _NT, Bblk), lambda i: (0, 0, i)),
            pl.BlockSpec((_NT, Bblk, _H), lambda i: (0, i, 0)),
        ],
        out_shape=[
            jax.ShapeDtypeStruct((_SEQ, _NT, B), f32),
            jax.ShapeDtypeStruct((_NT, B, _H), f32),
        ],
        scratch_shapes=[pltpu.VMEM((_SEQ, Bblk, _H), f32)],
    )(idx3, taskT, p, q, emb_table.astype(f32), wet, wat, whht,
      bih, bhh, pw, pb)

    outs = jnp.transpose(outs_raw, (2, 0, 1))
    hidden = jnp.transpose(hid_raw, (1, 0, 2))
    return outs, hidden


# speedup vs baseline: 1.6423x; 1.4288x over previous
import jax
import jax.numpy as jnp
from jax.experimental import pallas as pl
from jax.experimental.pallas import tpu as pltpu

_NT = 27
_H = 128
_SEQ = 20
_NE = _NT * 3


def _gkt_kernel(taskc_ref, idx3c_ref, taskt_ref, emb_ref, wet_ref, wat_ref,
                whht_ref, bih_ref, bhh_ref, pw_ref, pb_ref, outs_ref, hid_ref):
    B = taskc_ref.shape[0]
    f32 = jnp.float32
    bf16 = jnp.bfloat16

    gi_tab = (jnp.dot(emb_ref[...], wet_ref[...],
                      preferred_element_type=f32) + bih_ref[...]).astype(bf16)
    wat_s = (wat_ref[...] * f32(1.0 / 28.0)).astype(bf16)
    whht = whht_ref[...].astype(bf16)
    bhh = bhh_ref[...]
    pw = pw_ref[...]
    pb = pb_ref[0, 0]

    iota81 = jax.lax.broadcasted_iota(jnp.int32, (B, _NE), 1)
    iota27l = jax.lax.broadcasted_iota(jnp.int32, (_NT, B), 0)

    zero_plane = jnp.zeros((B, _H), bf16)
    for n in range(_NT):
        hid_ref[n] = zero_plane

    S = jnp.zeros((B, _H), f32)
    dT = jnp.full((_NT, B), pb, f32)
    prev_h = jnp.zeros((B, _H), f32)

    col_masks = [taskc_ref[:, t:t + 1] for t in range(_SEQ)]

    for t in range(_SEQ):
        idx3c = idx3c_ref[:, t:t + 1]

        oh81 = (idx3c == iota81).astype(bf16)
        gi_e = jnp.dot(oh81, gi_tab, preferred_element_type=f32)

        gi = gi_e + jnp.dot((S + prev_h).astype(bf16), wat_s,
                            preferred_element_type=f32)
        gh = jnp.dot(prev_h.astype(bf16), whht,
                     preferred_element_type=f32) + bhh

        r = jax.nn.sigmoid(gi[:, :_H] + gh[:, :_H])
        z = jax.nn.sigmoid(gi[:, _H:2 * _H] + gh[:, _H:2 * _H])
        nn = jnp.tanh(gi[:, 2 * _H:] + r * gh[:, 2 * _H:])
        new_h = nn + z * (prev_h - nn)
        new_hb = new_h.astype(bf16)

        taskc = col_masks[t]
        next_h = zero_plane
        for n in range(_NT):
            old = hid_ref[n]
            upd = jnp.where(taskc == n, new_hb, old)
            hid_ref[n] = upd
            if t + 1 < _SEQ:
                next_h = next_h + jnp.where(col_masks[t + 1] == n, upd, bf16(0))

        S = S + new_h - prev_h
        prev_h = next_h.astype(f32)

        lnewT = jax.lax.dot_general(pw, new_h, (((1,), (1,)), ((), ())),
                                    preferred_element_type=f32) + pb
        taskt = taskt_ref[t:t + 1, :]
        dT = jnp.where(iota27l == taskt, lnewT, dT)
        outs_ref[t] = dT


def kernel(task_seq, status_seq, emb_table, gru_Wih, gru_Whh, gru_bih,
           gru_bhh, pred_W, pred_b):
    B = task_seq.shape[0]
    f32 = jnp.float32

    idx3 = task_seq * 3 + status_seq
    taskT = jnp.transpose(task_seq)
    wet = jnp.transpose(gru_Wih[:, :_H])
    wat = jnp.transpose(gru_Wih[:, _H:])
    whht = jnp.transpose(gru_Whh)
    bih = gru_bih.reshape(1, 3 * _H).astype(f32)
    bhh = gru_bhh.reshape(1, 3 * _H).astype(f32)
    pw = pred_W.reshape(1, _H).astype(f32)
    pb = pred_b.reshape(1, 1).astype(f32)

    outs_raw, hid_raw = pl.pallas_call(
        _gkt_kernel,
        out_shape=[
            jax.ShapeDtypeStruct((_SEQ, _NT, B), f32),
            jax.ShapeDtypeStruct((_NT, B, _H), jnp.bfloat16),
        ],
    )(task_seq, idx3, taskT, emb_table.astype(f32), wet, wat, whht,
      bih, bhh, pw, pb)

    outs = jnp.transpose(outs_raw, (2, 0, 1))
    hidden = jnp.transpose(hid_raw, (1, 0, 2)).astype(f32)
    return outs, hidden
